# baseline (speedup 1.0000x reference)
import functools

import jax
import jax.numpy as jnp
from jax import lax
from jax.experimental import pallas as pl
from jax.experimental.pallas import tpu as pltpu
from jax.experimental.pallas import tpu_sc as plsc

_L = 199
_D = 128
_NM = 32
_B = 1024
_NC = 2
_NS = 16
_NW = _NC * _NS
_BPW = _B // _NW
_GROUPS = tuple(range(0, 177, 16)) + (183,)


def _sc_body(xt_hbm, xp_hbm, masks_hbm, ind_hbm, out_hbm,
             xt_v, xp_v, out_v, masks_v, ind_v):
    wid = lax.axis_index("s") * _NC + lax.axis_index("c")
    pltpu.sync_copy(masks_hbm, masks_v)
    pltpu.sync_copy(ind_hbm, ind_v)
    lanes = lax.iota(jnp.int32, 16)

    def one_batch(b, carry):
        pltpu.sync_copy(xt_hbm.at[b], xt_v)
        pltpu.sync_copy(xp_hbm.at[b], xp_v)
        for g in _GROUPS:
            tok = g + lanes

            def step_a(d, c):
                dsp = jnp.full((16,), d, jnp.int32)
                v = plsc.load_gather(xt_v, [tok, dsp])
                better = v > c[0]
                return (jnp.where(better, v, c[0]),
                        jnp.where(better, d, c[1]))

            _, bestd = lax.fori_loop(
                0, _D, step_a,
                (jnp.full((16,), -3.4e38, jnp.float32),
                 jnp.zeros((16,), jnp.int32)), unroll=8)
            ix2 = plsc.load_gather(ind_v, [bestd])

            def step_b(d, s):
                dsp = jnp.full((16,), d, jnp.int32)
                m = plsc.load_gather(masks_v, [ix2, dsp])
                p = plsc.load_gather(xp_v, [tok, dsp])
                e = jnp.exp(p) * m
                plsc.store_scatter(out_v, [tok, dsp], e)
                return s + e

            s = lax.fori_loop(0, _D, step_b,
                              jnp.zeros((16,), jnp.float32), unroll=8)
            rinv = jnp.ones((16,), jnp.float32) / s

            def step_c(d, c):
                dsp = jnp.full((16,), d, jnp.int32)
                e = plsc.load_gather(out_v, [tok, dsp])
                plsc.store_scatter(out_v, [tok, dsp], e * rinv)
                return c

            lax.fori_loop(0, _D, step_c, 0, unroll=8)
        pltpu.sync_copy(out_v, out_hbm.at[b])
        return carry

    lax.fori_loop(wid * _BPW, (wid + 1) * _BPW, one_batch, 0)


def kernel(x_true, x_pred, masks, ind_of_ind):
    mesh = plsc.VectorSubcoreMesh(core_axis_name="c", subcore_axis_name="s")
    f = functools.partial(
        pl.kernel,
        mesh=mesh,
        compiler_params=pltpu.CompilerParams(needs_layout_passes=False),
        out_type=jax.ShapeDtypeStruct((_B, _L, _D), jnp.float32),
        scratch_types=[
            pltpu.VMEM((_L, _D), jnp.float32),
            pltpu.VMEM((_L, _D), jnp.float32),
            pltpu.VMEM((_L, _D), jnp.float32),
            pltpu.VMEM((_NM, _D), jnp.float32),
            pltpu.VMEM((_D,), jnp.int32),
        ],
    )(_sc_body)
    return f(x_true, x_pred, masks, ind_of_ind.astype(jnp.int32))


# speedup vs baseline: 5.7716x; 5.7716x over previous
import functools

import jax
import jax.numpy as jnp
from jax import lax
from jax.experimental import pallas as pl
from jax.experimental.pallas import tpu as pltpu
from jax.experimental.pallas import tpu_sc as plsc

_L = 199
_D = 128
_NM = 32
_B = 1024
_NC = 2
_NS = 16
_NW = _NC * _NS
_BPW = _B // _NW
_NCH = _D // 16


def _w_table_body(masks_ref, ind_ref, w_ref):
    ind = ind_ref[...]
    m_iota = lax.broadcasted_iota(jnp.int32, (_NM, _D), 0)
    sel = (ind == m_iota).astype(jnp.float32)
    w_ref[...] = lax.dot_general(sel, masks_ref[...],
                                 dimension_numbers=(((0,), (0,)), ((), ())),
                                 preferred_element_type=jnp.float32)


def _sc_body(xt_hbm, xp_hbm, w_hbm, out_hbm, xt_v, xp_v, out_v, w_v):
    wid = lax.axis_index("s") * _NC + lax.axis_index("c")
    pltpu.sync_copy(w_hbm, w_v)
    lanes = lax.iota(jnp.int32, 16)

    def one_token(t, carry):
        maxv = xt_v[t, pl.ds(0, 16)]
        cidx = jnp.zeros((16,), jnp.int32)
        for c in range(1, _NCH):
            v = xt_v[t, pl.ds(c * 16, 16)]
            better = v > maxv
            maxv = jnp.where(better, v, maxv)
            cidx = jnp.where(better, c, cidx)
        gmax = jax.lax.reduce_max(maxv, (0,))
        dcand = jnp.where(maxv == gmax, cidx * 16 + lanes, _D)
        bestd = jax.lax.reduce_min(dcand, (0,))
        es = []
        s = jnp.zeros((16,), jnp.float32)
        for c in range(_NCH):
            m = w_v[bestd, pl.ds(c * 16, 16)]
            p = xp_v[t, pl.ds(c * 16, 16)]
            e = jnp.exp(p) * m
            es.append(e)
            s = s + e
        total = jax.lax.reduce_sum(s, (0,))
        rinv = jnp.ones((16,), jnp.float32) / jnp.full((16,), total, jnp.float32)
        for c in range(_NCH):
            out_v[t, pl.ds(c * 16, 16)] = es[c] * rinv
        return carry

    def one_batch(b, carry):
        pltpu.sync_copy(xt_hbm.at[b], xt_v)
        pltpu.sync_copy(xp_hbm.at[b], xp_v)
        lax.fori_loop(0, _L, one_token, 0, unroll=2)
        pltpu.sync_copy(out_v, out_hbm.at[b])
        return carry

    lax.fori_loop(wid * _BPW, (wid + 1) * _BPW, one_batch, 0)


def kernel(x_true, x_pred, masks, ind_of_ind):
    w = pl.pallas_call(
        _w_table_body,
        out_shape=jax.ShapeDtypeStruct((_D, _D), jnp.float32),
    )(masks, ind_of_ind.astype(jnp.int32).reshape(1, _D))
    mesh = plsc.VectorSubcoreMesh(core_axis_name="c", subcore_axis_name="s")
    f = functools.partial(
        pl.kernel,
        mesh=mesh,
        compiler_params=pltpu.CompilerParams(needs_layout_passes=False),
        out_type=jax.ShapeDtypeStruct((_B, _L, _D), jnp.float32),
        scratch_types=[
            pltpu.VMEM((_L, _D), jnp.float32),
            pltpu.VMEM((_L, _D), jnp.float32),
            pltpu.VMEM((_L, _D), jnp.float32),
            pltpu.VMEM((_D, _D), jnp.float32),
        ],
    )(_sc_body)
    return f(x_true, x_pred, w)
